# Initial kernel scaffold; baseline (speedup 1.0000x reference)
import jax
import jax.numpy as jnp
from jax.experimental import pallas as pl


def kernel(atom_features, edge_index, r, lg_index, params):
    raise NotImplementedError("write your pallas kernel here")


# speedup vs baseline: 2.6404x; 2.6404x over previous
import functools

import jax
import jax.numpy as jnp
import numpy as np
from jax import lax
from jax.experimental import pallas as pl
from jax.experimental.pallas import tpu as pltpu
from jax.experimental.pallas import tpu_sc as plsc

N = 10000
DEG = 8
E = 80000
HID = 64
NC, NS = 2, 16
NW = NC * NS


def _sc_mesh():
    return plsc.VectorSubcoreMesh(
        core_axis_name="c", subcore_axis_name="s", num_cores=NC, num_subcores=NS)


_SC_PARAMS = pltpu.CompilerParams(use_tc_tiling_on_sc=False)


def _wsplit(B):
    if B == E:
        return 2560, 128
    if B == N:
        return 320, 80
    raise ValueError(B)


@functools.cache
def _make_gather(T, F, B):
    per, C = _wsplit(B)
    last = B - (NW - 1) * per
    n_full, n_last = per // C, last // C
    assert per % C == 0 and last % C == 0 and F % 16 == 0

    @functools.partial(
        pl.kernel,
        out_type=jax.ShapeDtypeStruct((B, F), jnp.float32),
        mesh=_sc_mesh(),
        compiler_params=_SC_PARAMS,
        scratch_types=[
            pltpu.VMEM((C,), jnp.int32),
            pltpu.VMEM((C, F), jnp.float32),
            pltpu.SemaphoreType.DMA,
        ],
    )
    def gk(table_hbm, idx_hbm, out_hbm, idx_v, rows_v, sem):
        wid = lax.axis_index("s") * NC + lax.axis_index("c")
        base = wid * per
        nch = jnp.where(wid == NW - 1, n_last, n_full)

        def body(i, carry):
            off = base + i * C
            pltpu.sync_copy(idx_hbm.at[pl.ds(off, C)], idx_v)
            pltpu.async_copy(table_hbm.at[idx_v], rows_v, sem).wait()
            pltpu.sync_copy(rows_v, out_hbm.at[pl.ds(off, C)])
            return carry

        lax.fori_loop(0, nch, body, 0)

    return gk


@functools.cache
def _make_scatter(T, F, B):
    per, C = _wsplit(B)
    last = B - (NW - 1) * per
    n_full, n_last = per // C, last // C
    Fc = min(F, 128)
    nfc = F // Fc
    TR = T // NS
    ZR = 125 if TR % 125 == 0 else TR
    assert TR % ZR == 0 and F % Fc == 0

    @functools.partial(
        pl.kernel,
        out_type=jax.ShapeDtypeStruct((NC, T, F), jnp.float32),
        mesh=_sc_mesh(),
        compiler_params=_SC_PARAMS,
        scratch_types=[
            pltpu.VMEM((C,), jnp.int32),
            pltpu.VMEM((C, Fc), jnp.float32),
            pltpu.VMEM((ZR, Fc), jnp.float32),
            pltpu.VMEM_SHARED((T, Fc), jnp.float32),
        ],
    )
    def sk(vals_hbm, idx_hbm, out_hbm, idx_v, vals_v, zz_v, acc):
        cid = lax.axis_index("c")
        sid = lax.axis_index("s")
        wid = sid * NC + cid
        base = wid * per
        nch = jnp.where(wid == NW - 1, n_last, n_full)

        def zrow(i, carry):
            def zcol(j, c2):
                zz_v[i, pl.ds(j * 16, 16)] = jnp.zeros((16,), jnp.float32)
                return c2
            return lax.fori_loop(0, Fc // 16, zcol, carry)

        lax.fori_loop(0, ZR, zrow, 0)

        for fc in range(nfc):
            def zb(i, carry):
                pltpu.sync_copy(zz_v, acc.at[pl.ds(sid * TR + i * ZR, ZR)])
                return carry

            lax.fori_loop(0, TR // ZR, zb, 0)
            plsc.subcore_barrier()

            def body(i, carry):
                off = base + i * C
                pltpu.sync_copy(idx_hbm.at[pl.ds(off, C)], idx_v)
                if nfc == 1:
                    pltpu.sync_copy(vals_hbm.at[pl.ds(off, C)], vals_v)
                else:
                    pltpu.sync_copy(
                        vals_hbm.at[pl.ds(off, C), pl.ds(fc * Fc, Fc)], vals_v)
                pltpu.sync_copy(vals_v, acc.at[idx_v], add=True)
                return carry

            lax.fori_loop(0, nch, body, 0)
            plsc.subcore_barrier()

            if nfc == 1:
                pltpu.sync_copy(
                    acc.at[pl.ds(sid * TR, TR)],
                    out_hbm.at[cid, pl.ds(sid * TR, TR)])
            else:
                pltpu.sync_copy(
                    acc.at[pl.ds(sid * TR, TR)],
                    out_hbm.at[cid, pl.ds(sid * TR, TR), pl.ds(fc * Fc, Fc)])
            plsc.subcore_barrier()

    return sk


def _sc_gather(table, idx):
    T, F = table.shape
    return _make_gather(T, F, idx.shape[0])(table, idx)


def _sc_scatter(vals, idx, T):
    B, F = vals.shape
    parts = _make_scatter(T, F, B)(vals, idx)
    return parts[0] + parts[1]


@functools.partial(jax.custom_vjp, nondiff_argnums=(2,))
def _gather(table, idx, T):
    return _sc_gather(table, idx)


def _gather_fwd(table, idx, T):
    return _sc_gather(table, idx), idx


def _gather_bwd(T, idx, g):
    return _sc_scatter(g, idx, T), None


_gather.defvjp(_gather_fwd, _gather_bwd)


@functools.partial(jax.custom_vjp, nondiff_argnums=(2,))
def _scatter(vals, idx, T):
    return _sc_scatter(vals, idx, T)


def _scatter_fwd(vals, idx, T):
    return _sc_scatter(vals, idx, T), idx


def _scatter_bwd(T, idx, g):
    return _sc_gather(g, idx), None


_scatter.defvjp(_scatter_fwd, _scatter_bwd)


def _mm_block(x_ref, w_ref, b_ref, o_ref):
    o_ref[...] = (
        jnp.dot(x_ref[...], w_ref[...], preferred_element_type=jnp.float32)
        + b_ref[...])


def _mm(x, w, b):
    R, K = x.shape
    Nc = w.shape[1]
    BR = 2000 if R <= N else 4000
    return pl.pallas_call(
        _mm_block,
        grid=(R // BR,),
        in_specs=[
            pl.BlockSpec((BR, K), lambda i: (i, 0)),
            pl.BlockSpec((K, Nc), lambda i: (0, 0)),
            pl.BlockSpec((1, Nc), lambda i: (0, 0)),
        ],
        out_specs=pl.BlockSpec((BR, Nc), lambda i: (i, 0)),
        out_shape=jax.ShapeDtypeStruct((R, Nc), jnp.float32),
    )(x, w, b)


@jax.custom_vjp
def _linear(x, w, b):
    return _mm(x, w, b)


def _linear_fwd(x, w, b):
    return _mm(x, w, b), (w,)


def _linear_bwd(res, g):
    (w,) = res
    dx = _mm(g, w.T, jnp.zeros((1, w.shape[0]), jnp.float32))
    return dx, jnp.zeros_like(w), jnp.zeros((1, w.shape[1]), jnp.float32)


_linear.defvjp(_linear_fwd, _linear_bwd)


def _lin(p, x):
    w = p['w']
    b = p['b'].reshape(1, -1) if 'b' in p else jnp.zeros((1, w.shape[1]), jnp.float32)
    return _linear(x, w, b)


def _rbf(x, vmin, vmax, bins):
    centers = jnp.linspace(vmin, vmax, bins)
    gamma = 1.0 / (centers[1] - centers[0])
    return jnp.exp(-gamma * (x[:, None] - centers) ** 2)


def _mlp(p1, p2, x):
    return jax.nn.silu(_lin(p2, jax.nn.silu(_lin(p1, x))))


def _rep8(v):
    return jnp.broadcast_to(v[:, None, :], (v.shape[0], DEG, v.shape[1])).reshape(
        v.shape[0] * DEG, v.shape[1])


def _egc_node(p, dst, x, y):
    e = (_rep8(_lin(p['src_gate'], x)) + _gather(_lin(p['dst_gate'], x), dst, N)
         + _lin(p['edge_gate'], y))
    sigma = jax.nn.sigmoid(e)
    Bh = _rep8(_lin(p['dst_update'], x))
    ssh = _scatter(sigma * Bh, dst, N)
    ss = _scatter(sigma, dst, N)
    h = ssh / (ss + 1e-6)
    x_new = x + jax.nn.silu(_lin(p['src_update'], x) + h)
    y_new = y + jax.nn.silu(e)
    return x_new, y_new


def _egc_edge(p, dst, m, z):
    A = _lin(p['src_gate'], m)
    Bm = _lin(p['dst_gate'], m)
    Bm_ld = _gather(Bm.reshape(N, DEG * HID), dst, N).reshape(E, DEG, HID)
    Cz = _lin(p['edge_gate'], z.reshape(E * DEG, HID)).reshape(E, DEG, HID)
    e = A[:, None, :] + Bm_ld + Cz
    sigma = jax.nn.sigmoid(e)
    Dm = _lin(p['dst_update'], m)
    vals = sigma * Dm[:, None, :]
    ssh = _scatter(vals.reshape(E, DEG * HID), dst, N).reshape(E, HID)
    ss = _scatter(sigma.reshape(E, DEG * HID), dst, N).reshape(E, HID)
    h = ssh / (ss + 1e-6)
    m_new = m + jax.nn.silu(_lin(p['src_update'], m) + h)
    z_new = z + jax.nn.silu(e)
    return m_new, z_new


def _cutoff(r):
    D, Rc = 0.1, 3.9
    c = jnp.where(r < Rc - D, jnp.ones_like(r),
                  0.5 - 0.5 * jnp.sin(np.pi * (r - Rc) / (2 * D)))
    return jnp.where(r > Rc + D, jnp.zeros_like(r), c)


def _forward(atom_features, dst, r, params):
    bl = jnp.linalg.norm(r, axis=1)
    y0 = _mlp(params['edge_mlp1'], params['edge_mlp2'], _rbf(bl, 0.0, 8.0, 80))

    rbl = jnp.concatenate([r, bl[:, None]], axis=1)
    r2bl = _gather(rbl.reshape(N, DEG * 4), dst, N).reshape(E, DEG, 4)
    r2, bl2 = r2bl[..., :3], r2bl[..., 3]
    cos = -jnp.sum(r[:, None, :] * r2, axis=-1) / (bl[:, None] * bl2)
    cos = jnp.clip(cos, -1.0, 1.0)
    z = _mlp(params['angle_mlp1'], params['angle_mlp2'],
             _rbf(cos.reshape(E * DEG), -1.0, 1.0, 40)).reshape(E, DEG, HID)

    x = _sc_gather(params['atom_emb'], atom_features)
    x0 = x
    y = y0
    for lp in params['alignn']:
        x, m = _egc_node(lp['node'], dst, x, y)
        y, z = _egc_edge(lp['edge'], dst, m, z)
    for lp in params['gcn']:
        x, y = _egc_node(lp, dst, x, y)

    bo_dst = _lin(params['bo_dst'], x)
    int_dst = _linear(x0, params['int_dst']['w'],
                      jnp.zeros((1, 4), jnp.float32))
    table16 = jnp.concatenate(
        [bo_dst, int_dst, jnp.zeros((N, 11), jnp.float32)], axis=1)
    g16 = _gather(table16, dst, N)

    bo = jax.nn.sigmoid(_rep8(_lin(params['bo_src'], x))
                        + g16[:, 0:1] + _lin(params['bo_edge'], y0))[:, 0]
    pp = jnp.exp(_rep8(_lin(params['int_src'], x0)) + g16[:, 1:5])
    f_rep = pp[:, 0] * jnp.exp(-pp[:, 1] * bl)
    f_att = pp[:, 2] * jnp.exp(-pp[:, 3] * bl)
    V = _cutoff(bl) * (f_rep - bo * f_att)
    V16 = jnp.pad(V[:, None], ((0, 0), (0, 15)))
    atomwise = _scatter(V16, dst, N)[:, 0]
    return jnp.mean(atomwise), atomwise


def kernel(atom_features, edge_index, r, lg_index, params):
    dst = edge_index[1]
    (energy, atomwise), dy_dr = jax.value_and_grad(
        lambda rr: _forward(atom_features, dst, rr, params), has_aux=True)(r)
    g16 = jnp.pad(-dy_dr, ((0, 0), (0, 13)))
    forces = _sc_scatter(g16, dst, N)[:, :3] * float(N)
    return energy, forces, atomwise


# speedup vs baseline: 2.6969x; 1.0214x over previous
import functools

import jax
import jax.numpy as jnp
import numpy as np
from jax import lax
from jax.experimental import pallas as pl
from jax.experimental.pallas import tpu as pltpu
from jax.experimental.pallas import tpu_sc as plsc

N = 10000
DEG = 8
E = 80000
HID = 64
NC, NS = 2, 16
NW = NC * NS


def _sc_mesh():
    return plsc.VectorSubcoreMesh(
        core_axis_name="c", subcore_axis_name="s", num_cores=NC, num_subcores=NS)


_SC_PARAMS = pltpu.CompilerParams(use_tc_tiling_on_sc=False)


def _wsplit(B, F):
    if B == E:
        return (2560, 64) if F > 128 else (2560, 128)
    if B == N:
        return 320, 80
    raise ValueError(B)


@functools.cache
def _make_gather(T, F, B):
    per, C = _wsplit(B, F)
    last = B - (NW - 1) * per
    n_full, n_last = per // C, last // C
    assert per % C == 0 and last % C == 0 and F % 16 == 0

    def pipeline(nch, cb, table_hbm, idx2_hbm, out_hbm, idx_v, rows, gsem, osem):
        pltpu.sync_copy(idx2_hbm.at[pl.ds(cb, nch)], idx_v.at[pl.ds(0, nch)])
        gd = [None, None]
        od = [None, None]
        for i in range(nch):
            b = i & 1
            if od[b] is not None:
                od[b].wait()
            gd[b] = pltpu.async_copy(table_hbm.at[idx_v.at[i]], rows[b], gsem[b])
            if i >= 1:
                pb = (i - 1) & 1
                gd[pb].wait()
                od[pb] = pltpu.async_copy(
                    rows[pb], out_hbm.at[pl.ds((cb + i - 1) * C, C)], osem[pb])
        lb = (nch - 1) & 1
        gd[lb].wait()
        od[lb] = pltpu.async_copy(
            rows[lb], out_hbm.at[pl.ds((cb + nch - 1) * C, C)], osem[lb])
        if nch >= 2 and od[1 - lb] is not None:
            od[1 - lb].wait()
        od[lb].wait()

    @functools.partial(
        pl.kernel,
        out_type=jax.ShapeDtypeStruct((B, F), jnp.float32),
        mesh=_sc_mesh(),
        compiler_params=_SC_PARAMS,
        scratch_types=[
            pltpu.VMEM((n_full, C), jnp.int32),
            pltpu.VMEM((C, F), jnp.float32),
            pltpu.VMEM((C, F), jnp.float32),
            pltpu.SemaphoreType.DMA,
            pltpu.SemaphoreType.DMA,
            pltpu.SemaphoreType.DMA,
            pltpu.SemaphoreType.DMA,
        ],
    )
    def gk(table_hbm, idx2_hbm, out_hbm, idx_v, rows0, rows1, g0, g1, o0, o1):
        wid = lax.axis_index("s") * NC + lax.axis_index("c")
        cb = wid * n_full

        @pl.when(wid == NW - 1)
        def _():
            pipeline(n_last, cb, table_hbm, idx2_hbm, out_hbm, idx_v,
                     [rows0, rows1], [g0, g1], [o0, o1])

        @pl.when(wid != NW - 1)
        def _():
            pipeline(n_full, cb, table_hbm, idx2_hbm, out_hbm, idx_v,
                     [rows0, rows1], [g0, g1], [o0, o1])

    return gk


@functools.cache
def _make_scatter(T, F, B):
    per, C = _wsplit(B, F)
    last = B - (NW - 1) * per
    n_full, n_last = per // C, last // C
    Fc = min(F, 128)
    nfc = F // Fc
    TR = T // NS
    ZR = 125 if TR % 125 == 0 else TR
    assert TR % ZR == 0 and F % Fc == 0

    def pipeline(nch, cb, vals_hbm, idx_v, acc, vbuf, vsem, ssem, fc):
        vd = [None, None]
        sd = [None, None]

        def src(i):
            if nfc == 1:
                return vals_hbm.at[pl.ds((cb + i) * C, C)]
            return vals_hbm.at[pl.ds((cb + i) * C, C), pl.ds(fc * Fc, Fc)]

        for i in range(nch):
            b = i & 1
            if sd[b] is not None:
                sd[b].wait()
            vd[b] = pltpu.async_copy(src(i), vbuf[b], vsem[b])
            if i >= 1:
                pb = (i - 1) & 1
                vd[pb].wait()
                sd[pb] = pltpu.async_copy(
                    vbuf[pb], acc.at[idx_v.at[i - 1]], ssem[pb], add=True)
        lb = (nch - 1) & 1
        vd[lb].wait()
        sd[lb] = pltpu.async_copy(
            vbuf[lb], acc.at[idx_v.at[nch - 1]], ssem[lb], add=True)
        if nch >= 2 and sd[1 - lb] is not None:
            sd[1 - lb].wait()
        sd[lb].wait()

    @functools.partial(
        pl.kernel,
        out_type=jax.ShapeDtypeStruct((NC, T, F), jnp.float32),
        mesh=_sc_mesh(),
        compiler_params=_SC_PARAMS,
        scratch_types=[
            pltpu.VMEM((n_full, C), jnp.int32),
            pltpu.VMEM((C, Fc), jnp.float32),
            pltpu.VMEM((C, Fc), jnp.float32),
            pltpu.VMEM((ZR, Fc), jnp.float32),
            pltpu.VMEM_SHARED((T, Fc), jnp.float32),
            pltpu.SemaphoreType.DMA,
            pltpu.SemaphoreType.DMA,
            pltpu.SemaphoreType.DMA,
            pltpu.SemaphoreType.DMA,
        ],
    )
    def sk(vals_hbm, idx2_hbm, out_hbm, idx_v, v0, v1, zz_v, acc,
           vs0, vs1, ss0, ss1):
        cid = lax.axis_index("c")
        sid = lax.axis_index("s")
        wid = sid * NC + cid
        cb = wid * n_full

        def zrow(i, carry):
            def zcol(j, c2):
                zz_v[i, pl.ds(j * 16, 16)] = jnp.zeros((16,), jnp.float32)
                return c2
            return lax.fori_loop(0, Fc // 16, zcol, carry)

        lax.fori_loop(0, ZR, zrow, 0)

        @pl.when(wid == NW - 1)
        def _():
            pltpu.sync_copy(idx2_hbm.at[pl.ds(cb, n_last)],
                            idx_v.at[pl.ds(0, n_last)])

        @pl.when(wid != NW - 1)
        def _():
            pltpu.sync_copy(idx2_hbm.at[pl.ds(cb, n_full)], idx_v)

        for fc in range(nfc):
            def zb(i, carry):
                pltpu.sync_copy(zz_v, acc.at[pl.ds(sid * TR + i * ZR, ZR)])
                return carry

            lax.fori_loop(0, TR // ZR, zb, 0)
            plsc.subcore_barrier()

            @pl.when(wid == NW - 1)
            def _():
                pipeline(n_last, cb, vals_hbm, idx_v, acc, [v0, v1],
                         [vs0, vs1], [ss0, ss1], fc)

            @pl.when(wid != NW - 1)
            def _():
                pipeline(n_full, cb, vals_hbm, idx_v, acc, [v0, v1],
                         [vs0, vs1], [ss0, ss1], fc)

            plsc.subcore_barrier()

            if nfc == 1:
                pltpu.sync_copy(
                    acc.at[pl.ds(sid * TR, TR)],
                    out_hbm.at[cid, pl.ds(sid * TR, TR)])
            else:
                pltpu.sync_copy(
                    acc.at[pl.ds(sid * TR, TR)],
                    out_hbm.at[cid, pl.ds(sid * TR, TR), pl.ds(fc * Fc, Fc)])
            plsc.subcore_barrier()

    return sk


def _sc_gather(table, idx):
    T, F = table.shape
    B = idx.shape[0]
    _, C = _wsplit(B, F)
    return _make_gather(T, F, B)(table, idx.reshape(B // C, C))


def _sc_scatter(vals, idx, T):
    B, F = vals.shape
    _, C = _wsplit(B, F)
    parts = _make_scatter(T, F, B)(vals, idx.reshape(B // C, C))
    return parts[0] + parts[1]


@functools.partial(jax.custom_vjp, nondiff_argnums=(2,))
def _gather(table, idx, T):
    return _sc_gather(table, idx)


def _gather_fwd(table, idx, T):
    return _sc_gather(table, idx), idx


def _gather_bwd(T, idx, g):
    return _sc_scatter(g, idx, T), None


_gather.defvjp(_gather_fwd, _gather_bwd)


@functools.partial(jax.custom_vjp, nondiff_argnums=(2,))
def _scatter(vals, idx, T):
    return _sc_scatter(vals, idx, T)


def _scatter_fwd(vals, idx, T):
    return _sc_scatter(vals, idx, T), idx


def _scatter_bwd(T, idx, g):
    return _sc_gather(g, idx), None


_scatter.defvjp(_scatter_fwd, _scatter_bwd)


def _mm_block(x_ref, w_ref, b_ref, o_ref):
    o_ref[...] = (
        jnp.dot(x_ref[...], w_ref[...], preferred_element_type=jnp.float32)
        + b_ref[...])


def _mm(x, w, b):
    R, K = x.shape
    Nc = w.shape[1]
    BR = 2000 if R <= N else 4000
    return pl.pallas_call(
        _mm_block,
        grid=(R // BR,),
        in_specs=[
            pl.BlockSpec((BR, K), lambda i: (i, 0)),
            pl.BlockSpec((K, Nc), lambda i: (0, 0)),
            pl.BlockSpec((1, Nc), lambda i: (0, 0)),
        ],
        out_specs=pl.BlockSpec((BR, Nc), lambda i: (i, 0)),
        out_shape=jax.ShapeDtypeStruct((R, Nc), jnp.float32),
    )(x, w, b)


@jax.custom_vjp
def _linear(x, w, b):
    return _mm(x, w, b)


def _linear_fwd(x, w, b):
    return _mm(x, w, b), (w,)


def _linear_bwd(res, g):
    (w,) = res
    dx = _mm(g, w.T, jnp.zeros((1, w.shape[0]), jnp.float32))
    return dx, jnp.zeros_like(w), jnp.zeros((1, w.shape[1]), jnp.float32)


_linear.defvjp(_linear_fwd, _linear_bwd)


def _lin(p, x):
    w = p['w']
    b = p['b'].reshape(1, -1) if 'b' in p else jnp.zeros((1, w.shape[1]), jnp.float32)
    return _linear(x, w, b)


def _rbf(x, vmin, vmax, bins):
    centers = jnp.linspace(vmin, vmax, bins)
    gamma = 1.0 / (centers[1] - centers[0])
    return jnp.exp(-gamma * (x[:, None] - centers) ** 2)


def _mlp(p1, p2, x):
    return jax.nn.silu(_lin(p2, jax.nn.silu(_lin(p1, x))))


def _rep8(v):
    return jnp.broadcast_to(v[:, None, :], (v.shape[0], DEG, v.shape[1])).reshape(
        v.shape[0] * DEG, v.shape[1])


def _egc_node(p, dst, x, y):
    e = (_rep8(_lin(p['src_gate'], x)) + _gather(_lin(p['dst_gate'], x), dst, N)
         + _lin(p['edge_gate'], y))
    sigma = jax.nn.sigmoid(e)
    Bh = _rep8(_lin(p['dst_update'], x))
    ssh = _scatter(sigma * Bh, dst, N)
    ss = _scatter(sigma, dst, N)
    h = ssh / (ss + 1e-6)
    x_new = x + jax.nn.silu(_lin(p['src_update'], x) + h)
    y_new = y + jax.nn.silu(e)
    return x_new, y_new


def _egc_edge(p, dst, m, z):
    A = _lin(p['src_gate'], m)
    Bm = _lin(p['dst_gate'], m)
    Bm_ld = _gather(Bm.reshape(N, DEG * HID), dst, N).reshape(E, DEG, HID)
    Cz = _lin(p['edge_gate'], z.reshape(E * DEG, HID)).reshape(E, DEG, HID)
    e = A[:, None, :] + Bm_ld + Cz
    sigma = jax.nn.sigmoid(e)
    Dm = _lin(p['dst_update'], m)
    vals = sigma * Dm[:, None, :]
    ssh = _scatter(vals.reshape(E, DEG * HID), dst, N).reshape(E, HID)
    ss = _scatter(sigma.reshape(E, DEG * HID), dst, N).reshape(E, HID)
    h = ssh / (ss + 1e-6)
    m_new = m + jax.nn.silu(_lin(p['src_update'], m) + h)
    z_new = z + jax.nn.silu(e)
    return m_new, z_new


def _cutoff(r):
    D, Rc = 0.1, 3.9
    c = jnp.where(r < Rc - D, jnp.ones_like(r),
                  0.5 - 0.5 * jnp.sin(np.pi * (r - Rc) / (2 * D)))
    return jnp.where(r > Rc + D, jnp.zeros_like(r), c)


def _forward(atom_features, dst, r, params):
    bl = jnp.linalg.norm(r, axis=1)
    y0 = _mlp(params['edge_mlp1'], params['edge_mlp2'], _rbf(bl, 0.0, 8.0, 80))

    rbl = jnp.concatenate([r, bl[:, None]], axis=1)
    r2bl = _gather(rbl.reshape(N, DEG * 4), dst, N).reshape(E, DEG, 4)
    r2, bl2 = r2bl[..., :3], r2bl[..., 3]
    cos = -jnp.sum(r[:, None, :] * r2, axis=-1) / (bl[:, None] * bl2)
    cos = jnp.clip(cos, -1.0, 1.0)
    z = _mlp(params['angle_mlp1'], params['angle_mlp2'],
             _rbf(cos.reshape(E * DEG), -1.0, 1.0, 40)).reshape(E, DEG, HID)

    x = _sc_gather(params['atom_emb'], atom_features)
    x0 = x
    y = y0
    for lp in params['alignn']:
        x, m = _egc_node(lp['node'], dst, x, y)
        y, z = _egc_edge(lp['edge'], dst, m, z)
    for lp in params['gcn']:
        x, y = _egc_node(lp, dst, x, y)

    bo_dst = _lin(params['bo_dst'], x)
    int_dst = _linear(x0, params['int_dst']['w'],
                      jnp.zeros((1, 4), jnp.float32))
    table16 = jnp.concatenate(
        [bo_dst, int_dst, jnp.zeros((N, 11), jnp.float32)], axis=1)
    g16 = _gather(table16, dst, N)

    bo = jax.nn.sigmoid(_rep8(_lin(params['bo_src'], x))
                        + g16[:, 0:1] + _lin(params['bo_edge'], y0))[:, 0]
    pp = jnp.exp(_rep8(_lin(params['int_src'], x0)) + g16[:, 1:5])
    f_rep = pp[:, 0] * jnp.exp(-pp[:, 1] * bl)
    f_att = pp[:, 2] * jnp.exp(-pp[:, 3] * bl)
    V = _cutoff(bl) * (f_rep - bo * f_att)
    V16 = jnp.pad(V[:, None], ((0, 0), (0, 15)))
    atomwise = _scatter(V16, dst, N)[:, 0]
    return jnp.mean(atomwise), atomwise


def kernel(atom_features, edge_index, r, lg_index, params):
    dst = edge_index[1]
    (energy, atomwise), dy_dr = jax.value_and_grad(
        lambda rr: _forward(atom_features, dst, rr, params), has_aux=True)(r)
    g16 = jnp.pad(-dy_dr, ((0, 0), (0, 13)))
    forces = _sc_scatter(g16, dst, N)[:, :3] * float(N)
    return energy, forces, atomwise


# speedup vs baseline: 4.2561x; 1.5782x over previous
import functools

import jax
import jax.numpy as jnp
import numpy as np
from jax import lax
from jax.experimental import pallas as pl
from jax.experimental.pallas import tpu as pltpu
from jax.experimental.pallas import tpu_sc as plsc

N = 10000
DEG = 8
E = 80000
HID = 64
NC, NS = 2, 16
NW = NC * NS


def _sc_mesh():
    return plsc.VectorSubcoreMesh(
        core_axis_name="c", subcore_axis_name="s", num_cores=NC, num_subcores=NS)


_SC_PARAMS = pltpu.CompilerParams(use_tc_tiling_on_sc=False)


def _wsplit(B, F):
    if B == E:
        return (2560, 64) if F > 128 else (2560, 128)
    if B == N:
        return 320, 80
    raise ValueError(B)


@functools.cache
def _make_gather(T, F, B):
    per, C = _wsplit(B, F)
    last = B - (NW - 1) * per
    n_full, n_last = per // C, last // C
    tiled = (F % 128 == 0)
    assert per % C == 0 and last % C == 0 and F % 16 == 0

    def pipeline(nch, wid, cb, table_hbm, idx3_hbm, out_hbm, idx_v, rows,
                 gsem, osem):
        pltpu.sync_copy(idx3_hbm.at[wid], idx_v)
        gd = [None, None]
        od = [None, None]
        for i in range(nch):
            b = i & 1
            if od[b] is not None:
                od[b].wait()
            gd[b] = pltpu.async_copy(table_hbm.at[idx_v.at[i]], rows[b], gsem[b])
            if i >= 1:
                pb = (i - 1) & 1
                gd[pb].wait()
                od[pb] = pltpu.async_copy(
                    rows[pb], out_hbm.at[pl.ds((cb + i - 1) * C, C)], osem[pb])
        lb = (nch - 1) & 1
        gd[lb].wait()
        od[lb] = pltpu.async_copy(
            rows[lb], out_hbm.at[pl.ds((cb + nch - 1) * C, C)], osem[lb])
        if nch >= 2 and od[1 - lb] is not None:
            od[1 - lb].wait()
        od[lb].wait()

    @functools.partial(
        pl.kernel,
        out_type=jax.ShapeDtypeStruct((B, F), jnp.float32),
        mesh=_sc_mesh(),
        compiler_params=None if tiled else _SC_PARAMS,
        scratch_types=[
            pltpu.VMEM((n_full, C), jnp.int32),
            pltpu.VMEM((C, F), jnp.float32),
            pltpu.VMEM((C, F), jnp.float32),
            pltpu.SemaphoreType.DMA,
            pltpu.SemaphoreType.DMA,
            pltpu.SemaphoreType.DMA,
            pltpu.SemaphoreType.DMA,
        ],
    )
    def gk(table_hbm, idx3_hbm, out_hbm, idx_v, rows0, rows1, g0, g1, o0, o1):
        wid = lax.axis_index("s") * NC + lax.axis_index("c")
        cb = wid * n_full

        @pl.when(wid == NW - 1)
        def _():
            pipeline(n_last, wid, cb, table_hbm, idx3_hbm, out_hbm, idx_v,
                     [rows0, rows1], [g0, g1], [o0, o1])

        @pl.when(wid != NW - 1)
        def _():
            pipeline(n_full, wid, cb, table_hbm, idx3_hbm, out_hbm, idx_v,
                     [rows0, rows1], [g0, g1], [o0, o1])

    return gk


@functools.cache
def _make_scatter(T, F, B):
    per, C = _wsplit(B, F)
    last = B - (NW - 1) * per
    n_full, n_last = per // C, last // C
    Fc = 64 if F > 128 else min(F, 128)
    nfc = F // Fc
    tiled = False
    TRa = 8 * ((T // NS + 7) // 8)
    TRl = T - (NS - 1) * TRa
    assert F % Fc == 0 and 0 < TRl <= TRa

    def pipeline(nch, cb, vals_hbm, idx_v, acc, vbuf, vsem, ssem, fc):
        vd = [None, None]
        sd = [None, None]

        def src(i):
            if nfc == 1:
                return vals_hbm.at[pl.ds((cb + i) * C, C)]
            return vals_hbm.at[pl.ds((cb + i) * C, C), pl.ds(fc * Fc, Fc)]

        for i in range(nch):
            b = i & 1
            if sd[b] is not None:
                sd[b].wait()
            vd[b] = pltpu.async_copy(src(i), vbuf[b], vsem[b])
            if i >= 1:
                pb = (i - 1) & 1
                vd[pb].wait()
                sd[pb] = pltpu.async_copy(
                    vbuf[pb], acc.at[idx_v.at[i - 1]], ssem[pb], add=True)
        lb = (nch - 1) & 1
        vd[lb].wait()
        sd[lb] = pltpu.async_copy(
            vbuf[lb], acc.at[idx_v.at[nch - 1]], ssem[lb], add=True)
        if nch >= 2 and sd[1 - lb] is not None:
            sd[1 - lb].wait()
        sd[lb].wait()

    @functools.partial(
        pl.kernel,
        out_type=jax.ShapeDtypeStruct((NC, T, F), jnp.float32),
        mesh=_sc_mesh(),
        compiler_params=None if tiled else _SC_PARAMS,
        scratch_types=[
            pltpu.VMEM((n_full, C), jnp.int32),
            pltpu.VMEM((C, Fc), jnp.float32),
            pltpu.VMEM((C, Fc), jnp.float32),
            pltpu.VMEM((TRa, Fc), jnp.float32),
            pltpu.VMEM_SHARED((T, Fc), jnp.float32),
            pltpu.SemaphoreType.DMA,
            pltpu.SemaphoreType.DMA,
            pltpu.SemaphoreType.DMA,
            pltpu.SemaphoreType.DMA,
        ],
    )
    def sk(vals_hbm, idx3_hbm, out_hbm, idx_v, v0, v1, zz_v, acc,
           vs0, vs1, ss0, ss1):
        cid = lax.axis_index("c")
        sid = lax.axis_index("s")
        wid = sid * NC + cid
        cb = wid * n_full

        def zrow(i, carry):
            def zcol(j, c2):
                zz_v[i, pl.ds(j * 16, 16)] = jnp.zeros((16,), jnp.float32)
                return c2
            return lax.fori_loop(0, Fc // 16, zcol, carry)

        lax.fori_loop(0, TRa, zrow, 0)

        pltpu.sync_copy(idx3_hbm.at[wid], idx_v)

        for fc in range(nfc):
            @pl.when(sid == NS - 1)
            def _():
                pltpu.sync_copy(zz_v.at[pl.ds(0, TRl)],
                                acc.at[pl.ds((NS - 1) * TRa, TRl)])

            @pl.when(sid != NS - 1)
            def _():
                pltpu.sync_copy(zz_v, acc.at[pl.ds(sid * TRa, TRa)])

            plsc.subcore_barrier()

            @pl.when(wid == NW - 1)
            def _():
                pipeline(n_last, cb, vals_hbm, idx_v, acc, [v0, v1],
                         [vs0, vs1], [ss0, ss1], fc)

            @pl.when(wid != NW - 1)
            def _():
                pipeline(n_full, cb, vals_hbm, idx_v, acc, [v0, v1],
                         [vs0, vs1], [ss0, ss1], fc)

            plsc.subcore_barrier()

            def flush(r0, nr):
                if nfc == 1:
                    pltpu.sync_copy(acc.at[pl.ds(r0, nr)],
                                    out_hbm.at[cid, pl.ds(r0, nr)])
                else:
                    pltpu.sync_copy(
                        acc.at[pl.ds(r0, nr)],
                        out_hbm.at[cid, pl.ds(r0, nr), pl.ds(fc * Fc, Fc)])

            @pl.when(sid == NS - 1)
            def _():
                flush((NS - 1) * TRa, TRl)

            @pl.when(sid != NS - 1)
            def _():
                flush(sid * TRa, TRa)

            plsc.subcore_barrier()

    return sk


def _idx3(idx, B, F):
    per, C = _wsplit(B, F)
    return jnp.pad(idx, (0, NW * per - B)).reshape(NW, per // C, C)


def _sc_gather(table, idx):
    T, F = table.shape
    B = idx.shape[0]
    return _make_gather(T, F, B)(table, _idx3(idx, B, F))


def _sc_scatter(vals, idx, T):
    B, F = vals.shape
    parts = _make_scatter(T, F, B)(vals, _idx3(idx, B, F))
    return parts[0] + parts[1]


@functools.partial(jax.custom_vjp, nondiff_argnums=(2,))
def _gather(table, idx, T):
    return _sc_gather(table, idx)


def _gather_fwd(table, idx, T):
    return _sc_gather(table, idx), idx


def _gather_bwd(T, idx, g):
    return _sc_scatter(g, idx, T), None


_gather.defvjp(_gather_fwd, _gather_bwd)


@functools.partial(jax.custom_vjp, nondiff_argnums=(2,))
def _scatter(vals, idx, T):
    return _sc_scatter(vals, idx, T)


def _scatter_fwd(vals, idx, T):
    return _sc_scatter(vals, idx, T), idx


def _scatter_bwd(T, idx, g):
    return _sc_gather(g, idx), None


_scatter.defvjp(_scatter_fwd, _scatter_bwd)


def _mm_block(x_ref, w_ref, b_ref, o_ref):
    o_ref[...] = (
        jnp.dot(x_ref[...], w_ref[...], preferred_element_type=jnp.float32)
        + b_ref[...])


def _mm(x, w, b):
    R, K = x.shape
    Nc = w.shape[1]
    BR = 2000 if R <= N else 4000
    return pl.pallas_call(
        _mm_block,
        grid=(R // BR,),
        in_specs=[
            pl.BlockSpec((BR, K), lambda i: (i, 0)),
            pl.BlockSpec((K, Nc), lambda i: (0, 0)),
            pl.BlockSpec((1, Nc), lambda i: (0, 0)),
        ],
        out_specs=pl.BlockSpec((BR, Nc), lambda i: (i, 0)),
        out_shape=jax.ShapeDtypeStruct((R, Nc), jnp.float32),
    )(x, w, b)


def _mm_g_block(x_ref, w_ref, b_ref, o_ref):
    for k in range(DEG):
        sl = pl.ds(k * HID, HID)
        o_ref[:, sl] = (
            jnp.dot(x_ref[:, sl], w_ref[...], preferred_element_type=jnp.float32)
            + b_ref[...])


def _mm_g(x, w, b):
    R = x.shape[0]
    BR = 1000
    return pl.pallas_call(
        _mm_g_block,
        grid=(R // BR,),
        in_specs=[
            pl.BlockSpec((BR, DEG * HID), lambda i: (i, 0)),
            pl.BlockSpec((HID, HID), lambda i: (0, 0)),
            pl.BlockSpec((1, HID), lambda i: (0, 0)),
        ],
        out_specs=pl.BlockSpec((BR, DEG * HID), lambda i: (i, 0)),
        out_shape=jax.ShapeDtypeStruct((R, DEG * HID), jnp.float32),
    )(x, w, b)


@jax.custom_vjp
def _linear_g(x, w, b):
    return _mm_g(x, w, b)


def _linear_g_fwd(x, w, b):
    return _mm_g(x, w, b), (w,)


def _linear_g_bwd(res, g):
    (w,) = res
    dx = _mm_g(g, w.T, jnp.zeros((1, w.shape[0]), jnp.float32))
    return dx, jnp.zeros_like(w), jnp.zeros((1, w.shape[1]), jnp.float32)


_linear_g.defvjp(_linear_g_fwd, _linear_g_bwd)


@jax.custom_vjp
def _linear(x, w, b):
    return _mm(x, w, b)


def _linear_fwd(x, w, b):
    return _mm(x, w, b), (w,)


def _linear_bwd(res, g):
    (w,) = res
    dx = _mm(g, w.T, jnp.zeros((1, w.shape[0]), jnp.float32))
    return dx, jnp.zeros_like(w), jnp.zeros((1, w.shape[1]), jnp.float32)


_linear.defvjp(_linear_fwd, _linear_bwd)


def _lin(p, x):
    w = p['w']
    b = p['b'].reshape(1, -1) if 'b' in p else jnp.zeros((1, w.shape[1]), jnp.float32)
    return _linear(x, w, b)


def _silu(u):
    return u * jax.nn.sigmoid(u)


def _dsilu(u):
    s = jax.nn.sigmoid(u)
    return s + u * s * (1.0 - s)


def _fmlp_stages(t, w1, b1, w2, b2, vmin, dv, gamma, bins):
    c = vmin + dv * lax.broadcasted_iota(jnp.int32, (1, bins), 1).astype(jnp.float32)
    phi = jnp.exp(-gamma * (t - c) ** 2)
    u1 = jnp.dot(phi, w1, preferred_element_type=jnp.float32) + b1
    a1 = _silu(u1)
    u2 = jnp.dot(a1, w2, preferred_element_type=jnp.float32) + b2
    return c, phi, u1, a1, u2


def _fmlp_fwd_block(vmin, dv, gamma, bins,
                    t_ref, w1_ref, b1_ref, w2_ref, b2_ref, o_ref):
    _, _, _, _, u2 = _fmlp_stages(t_ref[...], w1_ref[...], b1_ref[...],
                                  w2_ref[...], b2_ref[...], vmin, dv, gamma, bins)
    o_ref[...] = _silu(u2)


def _fmlp_bwd_block(vmin, dv, gamma, bins,
                    t_ref, g_ref, w1_ref, b1_ref, w2_ref, b2_ref, dt_ref):
    t = t_ref[...]
    c, phi, u1, a1, u2 = _fmlp_stages(t, w1_ref[...], b1_ref[...],
                                      w2_ref[...], b2_ref[...],
                                      vmin, dv, gamma, bins)
    du2 = g_ref[...] * _dsilu(u2)
    da1 = lax.dot_general(du2, w2_ref[...], (((1,), (1,)), ((), ())),
                          preferred_element_type=jnp.float32)
    du1 = da1 * _dsilu(u1)
    dphi = lax.dot_general(du1, w1_ref[...], (((1,), (1,)), ((), ())),
                           preferred_element_type=jnp.float32)
    dt_ref[...] = jnp.sum(dphi * phi * (-2.0 * gamma) * (t - c),
                          axis=1, keepdims=True)


@functools.partial(jax.custom_vjp, nondiff_argnums=(5, 6, 7))
def _fmlp(t, w1, b1, w2, b2, vmin, vmax, bins):
    R = t.shape[0]
    H = w1.shape[1]
    BR = 2000 if R <= N else 4000
    dv = (vmax - vmin) / (bins - 1)
    gamma = 1.0 / dv
    return pl.pallas_call(
        functools.partial(_fmlp_fwd_block, vmin, dv, gamma, bins),
        grid=(R // BR,),
        in_specs=[
            pl.BlockSpec((BR, 1), lambda i: (i, 0)),
            pl.BlockSpec((bins, H), lambda i: (0, 0)),
            pl.BlockSpec((1, H), lambda i: (0, 0)),
            pl.BlockSpec((H, H), lambda i: (0, 0)),
            pl.BlockSpec((1, H), lambda i: (0, 0)),
        ],
        out_specs=pl.BlockSpec((BR, H), lambda i: (i, 0)),
        out_shape=jax.ShapeDtypeStruct((R, H), jnp.float32),
    )(t, w1, b1, w2, b2)


def _fmlp_f(t, w1, b1, w2, b2, vmin, vmax, bins):
    return _fmlp(t, w1, b1, w2, b2, vmin, vmax, bins), (t, w1, b1, w2, b2)


def _fmlp_b(vmin, vmax, bins, res, g):
    t, w1, b1, w2, b2 = res
    R = t.shape[0]
    H = w1.shape[1]
    BR = 2000 if R <= N else 4000
    dv = (vmax - vmin) / (bins - 1)
    gamma = 1.0 / dv
    dt = pl.pallas_call(
        functools.partial(_fmlp_bwd_block, vmin, dv, gamma, bins),
        grid=(R // BR,),
        in_specs=[
            pl.BlockSpec((BR, 1), lambda i: (i, 0)),
            pl.BlockSpec((BR, H), lambda i: (i, 0)),
            pl.BlockSpec((bins, H), lambda i: (0, 0)),
            pl.BlockSpec((1, H), lambda i: (0, 0)),
            pl.BlockSpec((H, H), lambda i: (0, 0)),
            pl.BlockSpec((1, H), lambda i: (0, 0)),
        ],
        out_specs=pl.BlockSpec((BR, 1), lambda i: (i, 0)),
        out_shape=jax.ShapeDtypeStruct((R, 1), jnp.float32),
    )(t, g, w1, b1, w2, b2)
    return (dt, jnp.zeros_like(w1), jnp.zeros_like(b1),
            jnp.zeros_like(w2), jnp.zeros_like(b2))


_fmlp.defvjp(_fmlp_f, _fmlp_b)


def _rbf_mlp(p1, p2, t, vmin, vmax, bins):
    return _fmlp(t[:, None], p1['w'], p1['b'].reshape(1, -1),
                 p2['w'], p2['b'].reshape(1, -1), vmin, vmax, bins)


def _fmlp8_fwd_block(vmin, dv, gamma, bins,
                     t_ref, w1_ref, b1_ref, w2_ref, b2_ref, o_ref):
    for k in range(DEG):
        _, _, _, _, u2 = _fmlp_stages(
            t_ref[:, pl.ds(k, 1)], w1_ref[...], b1_ref[...],
            w2_ref[...], b2_ref[...], vmin, dv, gamma, bins)
        o_ref[:, pl.ds(k * HID, HID)] = _silu(u2)


def _fmlp8_bwd_block(vmin, dv, gamma, bins,
                     t_ref, g_ref, w1_ref, b1_ref, w2_ref, b2_ref, dt_ref):
    for k in range(DEG):
        t = t_ref[:, pl.ds(k, 1)]
        c, phi, u1, a1, u2 = _fmlp_stages(t, w1_ref[...], b1_ref[...],
                                          w2_ref[...], b2_ref[...],
                                          vmin, dv, gamma, bins)
        du2 = g_ref[:, pl.ds(k * HID, HID)] * _dsilu(u2)
        da1 = lax.dot_general(du2, w2_ref[...], (((1,), (1,)), ((), ())),
                              preferred_element_type=jnp.float32)
        du1 = da1 * _dsilu(u1)
        dphi = lax.dot_general(du1, w1_ref[...], (((1,), (1,)), ((), ())),
                               preferred_element_type=jnp.float32)
        dt_ref[:, pl.ds(k, 1)] = jnp.sum(dphi * phi * (-2.0 * gamma) * (t - c),
                                         axis=1, keepdims=True)


@functools.partial(jax.custom_vjp, nondiff_argnums=(5, 6, 7))
def _fmlp8(t, w1, b1, w2, b2, vmin, vmax, bins):
    R = t.shape[0]
    H = w1.shape[1]
    BR = 2000
    dv = (vmax - vmin) / (bins - 1)
    gamma = 1.0 / dv
    return pl.pallas_call(
        functools.partial(_fmlp8_fwd_block, vmin, dv, gamma, bins),
        grid=(R // BR,),
        in_specs=[
            pl.BlockSpec((BR, DEG), lambda i: (i, 0)),
            pl.BlockSpec((bins, H), lambda i: (0, 0)),
            pl.BlockSpec((1, H), lambda i: (0, 0)),
            pl.BlockSpec((H, H), lambda i: (0, 0)),
            pl.BlockSpec((1, H), lambda i: (0, 0)),
        ],
        out_specs=pl.BlockSpec((BR, DEG * H), lambda i: (i, 0)),
        out_shape=jax.ShapeDtypeStruct((R, DEG * H), jnp.float32),
    )(t, w1, b1, w2, b2)


def _fmlp8_f(t, w1, b1, w2, b2, vmin, vmax, bins):
    return _fmlp8(t, w1, b1, w2, b2, vmin, vmax, bins), (t, w1, b1, w2, b2)


def _fmlp8_b(vmin, vmax, bins, res, g):
    t, w1, b1, w2, b2 = res
    R = t.shape[0]
    H = w1.shape[1]
    BR = 2000
    dv = (vmax - vmin) / (bins - 1)
    gamma = 1.0 / dv
    dt = pl.pallas_call(
        functools.partial(_fmlp8_bwd_block, vmin, dv, gamma, bins),
        grid=(R // BR,),
        in_specs=[
            pl.BlockSpec((BR, DEG), lambda i: (i, 0)),
            pl.BlockSpec((BR, DEG * H), lambda i: (i, 0)),
            pl.BlockSpec((bins, H), lambda i: (0, 0)),
            pl.BlockSpec((1, H), lambda i: (0, 0)),
            pl.BlockSpec((H, H), lambda i: (0, 0)),
            pl.BlockSpec((1, H), lambda i: (0, 0)),
        ],
        out_specs=pl.BlockSpec((BR, DEG), lambda i: (i, 0)),
        out_shape=jax.ShapeDtypeStruct((R, DEG), jnp.float32),
    )(t, g, w1, b1, w2, b2)
    return (dt, jnp.zeros_like(w1), jnp.zeros_like(b1),
            jnp.zeros_like(w2), jnp.zeros_like(b2))


_fmlp8.defvjp(_fmlp8_f, _fmlp8_b)


def _rep8(v):
    return jnp.broadcast_to(v[:, None, :], (v.shape[0], DEG, v.shape[1])).reshape(
        v.shape[0] * DEG, v.shape[1])


def _egc_node(p, dst, x, y):
    e = (_rep8(_lin(p['src_gate'], x)) + _gather(_lin(p['dst_gate'], x), dst, N)
         + _lin(p['edge_gate'], y))
    sigma = jax.nn.sigmoid(e)
    Bh = _rep8(_lin(p['dst_update'], x))
    ssh = _scatter(sigma * Bh, dst, N)
    ss = _scatter(sigma, dst, N)
    h = ssh / (ss + 1e-6)
    x_new = x + jax.nn.silu(_lin(p['src_update'], x) + h)
    y_new = y + jax.nn.silu(e)
    return x_new, y_new


def _egc_edge(p, dst, m, z):
    A = _lin(p['src_gate'], m)
    Bm = _lin(p['dst_gate'], m)
    Bm_ld = _gather(Bm.reshape(N, DEG * HID), dst, N)
    gp = p['edge_gate']
    Cz = _linear_g(z, gp['w'], gp['b'].reshape(1, -1))
    e = jnp.tile(A, (1, DEG)) + Bm_ld + Cz
    sigma = jax.nn.sigmoid(e)
    Dm = _lin(p['dst_update'], m)
    vals = sigma * jnp.tile(Dm, (1, DEG))
    ssh = _scatter(vals, dst, N)
    ss = _scatter(sigma, dst, N)
    h = (ssh / (ss + 1e-6)).reshape(E, HID)
    m_new = m + jax.nn.silu(_lin(p['src_update'], m) + h)
    z_new = z + jax.nn.silu(e)
    return m_new, z_new


def _cutoff(r):
    D, Rc = 0.1, 3.9
    c = jnp.where(r < Rc - D, jnp.ones_like(r),
                  0.5 - 0.5 * jnp.sin(np.pi * (r - Rc) / (2 * D)))
    return jnp.where(r > Rc + D, jnp.zeros_like(r), c)


def _forward(atom_features, dst, r, params):
    bl = jnp.linalg.norm(r, axis=1)
    y0 = _rbf_mlp(params['edge_mlp1'], params['edge_mlp2'], bl, 0.0, 8.0, 80)

    rbl = jnp.concatenate([r, bl[:, None]], axis=1)
    r2bl = _gather(rbl.reshape(N, DEG * 4), dst, N).reshape(E, DEG, 4)
    r2, bl2 = r2bl[..., :3], r2bl[..., 3]
    cos = -jnp.sum(r[:, None, :] * r2, axis=-1) / (bl[:, None] * bl2)
    cos = jnp.clip(cos, -1.0, 1.0)
    z = _fmlp8(cos, params['angle_mlp1']['w'],
               params['angle_mlp1']['b'].reshape(1, -1),
               params['angle_mlp2']['w'],
               params['angle_mlp2']['b'].reshape(1, -1), -1.0, 1.0, 40)

    x = _sc_gather(params['atom_emb'], atom_features)
    x0 = x
    y = y0
    for lp in params['alignn']:
        x, m = _egc_node(lp['node'], dst, x, y)
        y, z = _egc_edge(lp['edge'], dst, m, z)
    for lp in params['gcn']:
        x, y = _egc_node(lp, dst, x, y)

    bo_dst = _lin(params['bo_dst'], x)
    int_dst = _linear(x0, params['int_dst']['w'],
                      jnp.zeros((1, 4), jnp.float32))
    table16 = jnp.concatenate(
        [bo_dst, int_dst, jnp.zeros((N, 11), jnp.float32)], axis=1)
    g16 = _gather(table16, dst, N)

    bo = jax.nn.sigmoid(_rep8(_lin(params['bo_src'], x))
                        + g16[:, 0:1] + _lin(params['bo_edge'], y0))[:, 0]
    pp = jnp.exp(_rep8(_lin(params['int_src'], x0)) + g16[:, 1:5])
    f_rep = pp[:, 0] * jnp.exp(-pp[:, 1] * bl)
    f_att = pp[:, 2] * jnp.exp(-pp[:, 3] * bl)
    V = _cutoff(bl) * (f_rep - bo * f_att)
    V16 = jnp.pad(V[:, None], ((0, 0), (0, 15)))
    atomwise = _scatter(V16, dst, N)[:, 0]
    return jnp.mean(atomwise), atomwise


def kernel(atom_features, edge_index, r, lg_index, params):
    dst = edge_index[1]
    (energy, atomwise), dy_dr = jax.value_and_grad(
        lambda rr: _forward(atom_features, dst, rr, params), has_aux=True)(r)
    g16 = jnp.pad(-dy_dr, ((0, 0), (0, 13)))
    forces = _sc_scatter(g16, dst, N)[:, :3] * float(N)
    return energy, forces, atomwise
